# baseline (speedup 1.0000x reference)
import functools

import jax
import jax.numpy as jnp
from jax import lax
from jax.experimental import pallas as pl
from jax.experimental.pallas import tpu as pltpu
from jax.experimental.pallas import tpu_sc as plsc

N_NODES = 10000
N_EDGES = 320000
D = 128

NC = 2
NS = 16
NW = NC * NS
EPW = N_EDGES // NW
CH = 80
NCHUNK = EPW // CH
NPASS = 5
CPP = NCHUNK // NPASS
ACC_ROWS = 10240
RPS = ACC_ROWS // NS
ROW_BLK = 5000
N_BLKS = N_NODES // ROW_BLK


def _tc_post_body(p0_ref, p1_ref, f_ref, w_ref, wr_ref, b_ref, br_ref,
                  y_ref, s_ref, s2_ref):
    i = pl.program_id(0)
    f = f_ref[...]
    res = jnp.dot(f, wr_ref[...], preferred_element_type=jnp.float32)
    r = jnp.maximum(res + br_ref[...], 0.0) + f
    agg = p0_ref[...] + p1_ref[...]
    gcn = jnp.dot(agg, w_ref[...], preferred_element_type=jnp.float32,
                  precision=lax.Precision.HIGHEST)
    yv = jnp.maximum(gcn + b_ref[...], 0.0) + r
    y_ref[...] = yv

    @pl.when(i == 0)
    def _():
        s_ref[...] = jnp.zeros_like(s_ref)
        s2_ref[...] = jnp.zeros_like(s2_ref)

    s_ref[...] += jnp.sum(yv, axis=0, keepdims=True)
    s2_ref[...] += jnp.sum(yv * yv, axis=0, keepdims=True)


def _tc_norm_body(y_ref, s_ref, s2_ref, g_ref, be_ref, o_ref):
    n = jnp.float32(N_NODES)
    mean = s_ref[...] / n
    var = s2_ref[...] / n - mean * mean
    inv = lax.rsqrt(var + 1e-5)
    o_ref[...] = (y_ref[...] - mean) * (inv * g_ref[...]) + be_ref[...]


def _sc_segment_sum(feats, src_rs, dst_rs):
    mesh = plsc.VectorSubcoreMesh(core_axis_name="c", subcore_axis_name="s")

    @functools.partial(
        pl.kernel,
        out_type=[jax.ShapeDtypeStruct((ACC_ROWS, D), jnp.float32)] * 2,
        mesh=mesh,
        scratch_types=[
            pltpu.VMEM((CPP, CH), jnp.int32),
            pltpu.VMEM((CPP, CH), jnp.int32),
            pltpu.VMEM((CH, D), jnp.float32),
            pltpu.VMEM((CH, D), jnp.float32),
            pltpu.VMEM((CH, D), jnp.float32),
            pltpu.VMEM_SHARED((ACC_ROWS, D), jnp.float32),
            pltpu.SemaphoreType.DMA,
            pltpu.SemaphoreType.DMA,
            pltpu.SemaphoreType.DMA,
        ],
    )
    def sc_kernel(f_hbm, src_hbm, dst_hbm, out0_hbm, out1_hbm,
                  src_v, dst_v, buf_a, buf_b, buf_c, acc,
                  sem_a, sem_b, sem_c):
        c = lax.axis_index("c")
        s = lax.axis_index("s")
        wid = c * NS + s
        bufs = (buf_a, buf_b, buf_c)
        sems = (sem_a, sem_b, sem_c)

        @pl.loop(0, CH)
        def _(rr):
            @pl.loop(0, D, step=16)
            def _(cc):
                buf_a[rr, pl.ds(cc, 16)] = jnp.zeros((16,), jnp.float32)

        @pl.loop(0, RPS // CH)
        def _(t):
            pltpu.make_async_copy(
                buf_a, acc.at[pl.ds(s * RPS + t * CH, CH)], sem_a).start()

        @pl.loop(0, RPS // CH)
        def _(t):
            pltpu.make_async_copy(
                buf_a, acc.at[pl.ds(s * RPS + t * CH, CH)], sem_a).wait()

        plsc.subcore_barrier()

        @pl.loop(0, NPASS)
        def _(p):
            pltpu.sync_copy(src_hbm.at[wid, p], src_v)
            pltpu.sync_copy(dst_hbm.at[wid, p], dst_v)

            for t in range(3):
                pltpu.make_async_copy(
                    f_hbm.at[src_v.at[t]], bufs[t], sems[t]).start()

            @pl.loop(0, CPP - 1, step=3)
            def _(j):
                for t in range(3):
                    jj = j + t
                    pltpu.make_async_copy(
                        f_hbm.at[src_v.at[jj]], bufs[t], sems[t]).wait()
                    pltpu.sync_copy(bufs[t], acc.at[dst_v.at[jj]], add=True)
                    if t == 0:
                        pltpu.make_async_copy(
                            f_hbm.at[src_v.at[jj + 3]], bufs[t],
                            sems[t]).start()
                    else:
                        @pl.when(j < CPP - 4)
                        def _():
                            pltpu.make_async_copy(
                                f_hbm.at[src_v.at[jj + 3]], bufs[t],
                                sems[t]).start()

            pltpu.make_async_copy(
                f_hbm.at[src_v.at[CPP - 1]], buf_a, sem_a).wait()
            pltpu.sync_copy(buf_a, acc.at[dst_v.at[CPP - 1]], add=True)

        plsc.subcore_barrier()

        def copy_out(out_ref):
            def oslc(t):
                return out_ref.at[pl.ds(s * RPS + t * CH, CH)]

            @pl.loop(0, RPS // CH, step=2)
            def _(t):
                @pl.when(t > 0)
                def _():
                    pltpu.make_async_copy(buf_a, oslc(t - 2), sem_a).wait()
                pltpu.sync_copy(acc.at[pl.ds(s * RPS + t * CH, CH)], buf_a)
                pltpu.make_async_copy(buf_a, oslc(t), sem_a).start()

                @pl.when(t > 0)
                def _():
                    pltpu.make_async_copy(buf_b, oslc(t - 1), sem_b).wait()
                pltpu.sync_copy(
                    acc.at[pl.ds(s * RPS + (t + 1) * CH, CH)], buf_b)
                pltpu.make_async_copy(buf_b, oslc(t + 1), sem_b).start()

            last = RPS // CH - 2
            pltpu.make_async_copy(buf_a, oslc(last), sem_a).wait()
            pltpu.make_async_copy(buf_b, oslc(last + 1), sem_b).wait()

        @pl.when(c == 0)
        def _():
            copy_out(out0_hbm)

        @pl.when(c == 1)
        def _():
            copy_out(out1_hbm)

    return sc_kernel(feats, src_rs, dst_rs)


@jax.jit
def kernel(edge_index, feats, W, b, W_res, b_res, gamma, beta):
    ei = edge_index.astype(jnp.int32)
    src_rs = ei[0].reshape(NW, NPASS, CPP, CH)
    dst_rs = ei[1].reshape(NW, NPASS, CPP, CH)

    b2 = b.reshape(1, D)
    br2 = b_res.reshape(1, D)
    g2 = gamma.reshape(1, D)
    be2 = beta.reshape(1, D)

    row_spec = pl.BlockSpec((ROW_BLK, D), lambda i: (i, 0))
    full_spec = pl.BlockSpec((D, D), lambda i: (0, 0))
    vec_spec = pl.BlockSpec((1, D), lambda i: (0, 0))

    p0, p1 = _sc_segment_sum(feats, src_rs, dst_rs)

    y, s, s2 = pl.pallas_call(
        _tc_post_body,
        grid=(N_BLKS,),
        in_specs=[row_spec, row_spec, row_spec, full_spec, full_spec,
                  vec_spec, vec_spec],
        out_specs=[row_spec, vec_spec, vec_spec],
        out_shape=[
            jax.ShapeDtypeStruct((N_NODES, D), jnp.float32),
            jax.ShapeDtypeStruct((1, D), jnp.float32),
            jax.ShapeDtypeStruct((1, D), jnp.float32),
        ],
    )(p0, p1, feats, W, W_res, b2, br2)

    out = pl.pallas_call(
        _tc_norm_body,
        grid=(N_BLKS,),
        in_specs=[row_spec, vec_spec, vec_spec, vec_spec, vec_spec],
        out_specs=row_spec,
        out_shape=jax.ShapeDtypeStruct((N_NODES, D), jnp.float32),
    )(y, s, s2, g2, be2)

    return out


# speedup vs baseline: 1.0074x; 1.0074x over previous
import functools

import jax
import jax.numpy as jnp
from jax import lax
from jax.experimental import pallas as pl
from jax.experimental.pallas import tpu as pltpu
from jax.experimental.pallas import tpu_sc as plsc

N_NODES = 10000
N_EDGES = 320000
D = 128

NC = 2
NS = 16
NW = NC * NS
EPW = N_EDGES // NW
CH = 80
NCHUNK = EPW // CH
NPASS = 5
CPP = NCHUNK // NPASS
ACC_ROWS = 10240
RPS = ACC_ROWS // NS
ROW_BLK = 2000
N_BLKS = N_NODES // ROW_BLK


def _tc_post_body(p0_ref, p1_ref, f_ref, w_ref, wr_ref, b_ref, br_ref,
                  y_ref, s_ref, s2_ref):
    i = pl.program_id(0)
    f = f_ref[...]
    res = jnp.dot(f, wr_ref[...], preferred_element_type=jnp.float32)
    r = jnp.maximum(res + br_ref[...], 0.0) + f
    agg = p0_ref[...] + p1_ref[...]
    gcn = jnp.dot(agg, w_ref[...], preferred_element_type=jnp.float32,
                  precision=lax.Precision.HIGHEST)
    yv = jnp.maximum(gcn + b_ref[...], 0.0) + r
    y_ref[...] = yv

    @pl.when(i == 0)
    def _():
        s_ref[...] = jnp.zeros_like(s_ref)
        s2_ref[...] = jnp.zeros_like(s2_ref)

    s_ref[...] += jnp.sum(yv, axis=0, keepdims=True)
    s2_ref[...] += jnp.sum(yv * yv, axis=0, keepdims=True)


def _tc_norm_body(y_ref, s_ref, s2_ref, g_ref, be_ref, o_ref):
    n = jnp.float32(N_NODES)
    mean = s_ref[...] / n
    var = s2_ref[...] / n - mean * mean
    inv = lax.rsqrt(var + 1e-5)
    o_ref[...] = (y_ref[...] - mean) * (inv * g_ref[...]) + be_ref[...]


def _sc_segment_sum(feats, src_rs, dst_rs):
    mesh = plsc.VectorSubcoreMesh(core_axis_name="c", subcore_axis_name="s")

    @functools.partial(
        pl.kernel,
        out_type=[jax.ShapeDtypeStruct((ACC_ROWS, D), jnp.float32)] * 2,
        mesh=mesh,
        scratch_types=[
            pltpu.VMEM((CPP, CH), jnp.int32),
            pltpu.VMEM((CPP, CH), jnp.int32),
            pltpu.VMEM((CH, D), jnp.float32),
            pltpu.VMEM((CH, D), jnp.float32),
            pltpu.VMEM((CH, D), jnp.float32),
            pltpu.VMEM_SHARED((ACC_ROWS, D), jnp.float32),
            pltpu.SemaphoreType.DMA,
            pltpu.SemaphoreType.DMA,
            pltpu.SemaphoreType.DMA,
        ],
    )
    def sc_kernel(f_hbm, src_hbm, dst_hbm, out0_hbm, out1_hbm,
                  src_v, dst_v, buf_a, buf_b, buf_c, acc,
                  sem_a, sem_b, sem_c):
        c = lax.axis_index("c")
        s = lax.axis_index("s")
        wid = c * NS + s
        bufs = (buf_a, buf_b, buf_c)
        sems = (sem_a, sem_b, sem_c)

        @pl.loop(0, CH)
        def _(rr):
            @pl.loop(0, D, step=16)
            def _(cc):
                buf_a[rr, pl.ds(cc, 16)] = jnp.zeros((16,), jnp.float32)

        @pl.loop(0, RPS // CH)
        def _(t):
            pltpu.make_async_copy(
                buf_a, acc.at[pl.ds(s * RPS + t * CH, CH)], sem_a).start()

        @pl.loop(0, RPS // CH)
        def _(t):
            pltpu.make_async_copy(
                buf_a, acc.at[pl.ds(s * RPS + t * CH, CH)], sem_a).wait()

        plsc.subcore_barrier()

        @pl.loop(0, NPASS)
        def _(p):
            pltpu.sync_copy(src_hbm.at[wid, p], src_v)
            pltpu.sync_copy(dst_hbm.at[wid, p], dst_v)

            for t in range(3):
                pltpu.make_async_copy(
                    f_hbm.at[src_v.at[t]], bufs[t], sems[t]).start()

            @pl.loop(0, CPP - 1, step=3)
            def _(j):
                for t in range(3):
                    jj = j + t
                    pltpu.make_async_copy(
                        f_hbm.at[src_v.at[jj]], bufs[t], sems[t]).wait()
                    pltpu.sync_copy(bufs[t], acc.at[dst_v.at[jj]], add=True)
                    if t == 0:
                        pltpu.make_async_copy(
                            f_hbm.at[src_v.at[jj + 3]], bufs[t],
                            sems[t]).start()
                    else:
                        @pl.when(j < CPP - 4)
                        def _():
                            pltpu.make_async_copy(
                                f_hbm.at[src_v.at[jj + 3]], bufs[t],
                                sems[t]).start()

            pltpu.make_async_copy(
                f_hbm.at[src_v.at[CPP - 1]], buf_a, sem_a).wait()
            pltpu.sync_copy(buf_a, acc.at[dst_v.at[CPP - 1]], add=True)

        plsc.subcore_barrier()

        def copy_out(out_ref):
            def oslc(t):
                return out_ref.at[pl.ds(s * RPS + t * CH, CH)]

            @pl.loop(0, RPS // CH, step=2)
            def _(t):
                @pl.when(t > 0)
                def _():
                    pltpu.make_async_copy(buf_a, oslc(t - 2), sem_a).wait()
                pltpu.sync_copy(acc.at[pl.ds(s * RPS + t * CH, CH)], buf_a)
                pltpu.make_async_copy(buf_a, oslc(t), sem_a).start()

                @pl.when(t > 0)
                def _():
                    pltpu.make_async_copy(buf_b, oslc(t - 1), sem_b).wait()
                pltpu.sync_copy(
                    acc.at[pl.ds(s * RPS + (t + 1) * CH, CH)], buf_b)
                pltpu.make_async_copy(buf_b, oslc(t + 1), sem_b).start()

            last = RPS // CH - 2
            pltpu.make_async_copy(buf_a, oslc(last), sem_a).wait()
            pltpu.make_async_copy(buf_b, oslc(last + 1), sem_b).wait()

        @pl.when(c == 0)
        def _():
            copy_out(out0_hbm)

        @pl.when(c == 1)
        def _():
            copy_out(out1_hbm)

    return sc_kernel(feats, src_rs, dst_rs)


@jax.jit
def kernel(edge_index, feats, W, b, W_res, b_res, gamma, beta):
    ei = edge_index.astype(jnp.int32)
    src_rs = ei[0].reshape(NW, NPASS, CPP, CH)
    dst_rs = ei[1].reshape(NW, NPASS, CPP, CH)

    b2 = b.reshape(1, D)
    br2 = b_res.reshape(1, D)
    g2 = gamma.reshape(1, D)
    be2 = beta.reshape(1, D)

    row_spec = pl.BlockSpec((ROW_BLK, D), lambda i: (i, 0))
    full_spec = pl.BlockSpec((D, D), lambda i: (0, 0))
    vec_spec = pl.BlockSpec((1, D), lambda i: (0, 0))

    p0, p1 = _sc_segment_sum(feats, src_rs, dst_rs)

    y, s, s2 = pl.pallas_call(
        _tc_post_body,
        grid=(N_BLKS,),
        in_specs=[row_spec, row_spec, row_spec, full_spec, full_spec,
                  vec_spec, vec_spec],
        out_specs=[row_spec, vec_spec, vec_spec],
        out_shape=[
            jax.ShapeDtypeStruct((N_NODES, D), jnp.float32),
            jax.ShapeDtypeStruct((1, D), jnp.float32),
            jax.ShapeDtypeStruct((1, D), jnp.float32),
        ],
    )(p0, p1, feats, W, W_res, b2, br2)

    out = pl.pallas_call(
        _tc_norm_body,
        grid=(N_BLKS,),
        in_specs=[row_spec, vec_spec, vec_spec, vec_spec, vec_spec],
        out_specs=row_spec,
        out_shape=jax.ShapeDtypeStruct((N_NODES, D), jnp.float32),
    )(y, s, s2, g2, be2)

    return out
